# baseline (speedup 1.0000x reference)
import jax
import jax.numpy as jnp
from jax import lax
from jax.experimental import pallas as pl
from jax.experimental.pallas import tpu as pltpu
from jax.experimental.pallas import tpu_sc as plsc

_HBLK = 8

_NW = 32
_CHUNK_ROWS = 504


def _tc_copy_body(k_ref, ok_ref):
    ok_ref[...] = k_ref[...]


def _tc_copy(x):
    B, H, L, D = x.shape
    spec = pl.BlockSpec((1, _HBLK, L, D), lambda b, h: (b, h, 0, 0))
    return pl.pallas_call(
        _tc_copy_body,
        grid=(B, H // _HBLK),
        in_specs=[spec],
        out_specs=spec,
        out_shape=jax.ShapeDtypeStruct(x.shape, x.dtype),
        compiler_params=pltpu.CompilerParams(
            dimension_semantics=("arbitrary", "arbitrary"),
        ),
    )(x)


def _sc_copy(x):
    shape = x.shape
    rows = x.size // 128
    x2 = x.reshape(rows, 128)
    rows_per_w = rows // _NW
    sizes = [_CHUNK_ROWS] * (rows_per_w // _CHUNK_ROWS)
    if rows_per_w % _CHUNK_ROWS:
        sizes.append(rows_per_w % _CHUNK_ROWS)
    offs = [0]
    for s in sizes[:-1]:
        offs.append(offs[-1] + s)
    n_chunks = len(sizes)

    mesh = plsc.VectorSubcoreMesh(core_axis_name="c", subcore_axis_name="s")

    def run(x2):
        @pl.kernel(
            out_type=jax.ShapeDtypeStruct((rows, 128), jnp.float32),
            mesh=mesh,
            scratch_types=[
                pltpu.VMEM((504, 128), jnp.float32),
                pltpu.VMEM((504, 128), jnp.float32),
                pltpu.SemaphoreType.DMA,
                pltpu.SemaphoreType.DMA,
                pltpu.SemaphoreType.DMA,
                pltpu.SemaphoreType.DMA,
            ],
        )
        def sc_copy_kernel(in_hbm, out_hbm, buf0, buf1, g0, g1, s0, s1):
            wid = lax.axis_index("s") * 2 + lax.axis_index("c")
            base = wid * rows_per_w
            bufs = (buf0, buf1)
            gsems = (g0, g1)
            ssems = (s0, s1)

            gathers = [None] * n_chunks
            scatters = [None] * n_chunks

            def src(c):
                return in_hbm.at[pl.ds(base + offs[c], sizes[c])]

            def dst(c):
                return out_hbm.at[pl.ds(base + offs[c], sizes[c])]

            def buf(c):
                b = bufs[c % 2]
                if sizes[c] == _CHUNK_ROWS:
                    return b
                return b.at[pl.ds(0, sizes[c])]

            gathers[0] = pltpu.async_copy(src(0), buf(0), gsems[0])
            for c in range(n_chunks):
                p = c % 2
                gathers[c].wait()
                if c >= 1:
                    scatters[c - 1].wait()
                if c + 1 < n_chunks:
                    gathers[c + 1] = pltpu.async_copy(
                        src(c + 1), buf(c + 1), gsems[1 - p]
                    )
                scatters[c] = pltpu.async_copy(buf(c), dst(c), ssems[p])
            scatters[n_chunks - 1].wait()

        return sc_copy_kernel(x2)

    return run(x2).reshape(shape)


def kernel(new_k, new_v):
    out_v = _sc_copy(new_v)
    out_k = _tc_copy(new_k)
    return (out_k, out_v)


# speedup vs baseline: 1.0422x; 1.0422x over previous
import jax
import jax.numpy as jnp
from jax import lax
from jax.experimental import pallas as pl
from jax.experimental.pallas import tpu as pltpu
from jax.experimental.pallas import tpu_sc as plsc

_HBLK = 8

_NW = 32
_CHUNK_ROWS = 480


def _tc_copy_body(k_ref, ok_ref):
    ok_ref[...] = k_ref[...]


def _tc_copy(x):
    B, H, L, D = x.shape
    spec = pl.BlockSpec((1, _HBLK, L, D), lambda b, h: (b, h, 0, 0))
    return pl.pallas_call(
        _tc_copy_body,
        grid=(B, H // _HBLK),
        in_specs=[spec],
        out_specs=spec,
        out_shape=jax.ShapeDtypeStruct(x.shape, x.dtype),
        compiler_params=pltpu.CompilerParams(
            dimension_semantics=("arbitrary", "arbitrary"),
        ),
    )(x)


def _sc_copy(x):
    shape = x.shape
    rows = x.size // 128
    x2 = x.reshape(rows, 128)
    rows_per_w = rows // _NW
    sizes = [_CHUNK_ROWS] * (rows_per_w // _CHUNK_ROWS)
    if rows_per_w % _CHUNK_ROWS:
        sizes.append(rows_per_w % _CHUNK_ROWS)
    offs = [0]
    for s in sizes[:-1]:
        offs.append(offs[-1] + s)
    n_chunks = len(sizes)

    mesh = plsc.VectorSubcoreMesh(core_axis_name="c", subcore_axis_name="s")

    def run(x2):
        @pl.kernel(
            out_type=jax.ShapeDtypeStruct((rows, 128), jnp.float32),
            mesh=mesh,
            scratch_types=[
                pltpu.VMEM_SHARED((16, 480, 128), jnp.float32),
                pltpu.VMEM_SHARED((16, 480, 128), jnp.float32),
                pltpu.SemaphoreType.DMA,
                pltpu.SemaphoreType.DMA,
                pltpu.SemaphoreType.DMA,
                pltpu.SemaphoreType.DMA,
            ],
        )
        def sc_copy_kernel(in_hbm, out_hbm, buf0, buf1, g0, g1, s0, s1):
            sid = lax.axis_index("s")
            wid = sid * 2 + lax.axis_index("c")
            base = wid * rows_per_w
            bufs = (buf0, buf1)
            gsems = (g0, g1)
            ssems = (s0, s1)

            gathers = [None] * n_chunks
            scatters = [None] * n_chunks

            def src(c):
                return in_hbm.at[pl.ds(base + offs[c], sizes[c])]

            def dst(c):
                return out_hbm.at[pl.ds(base + offs[c], sizes[c])]

            def buf(c):
                b = bufs[c % 2].at[sid]
                if sizes[c] == _CHUNK_ROWS:
                    return b
                return b.at[pl.ds(0, sizes[c])]

            gathers[0] = pltpu.async_copy(src(0), buf(0), gsems[0])
            for c in range(n_chunks):
                p = c % 2
                gathers[c].wait()
                if c >= 1:
                    scatters[c - 1].wait()
                if c + 1 < n_chunks:
                    gathers[c + 1] = pltpu.async_copy(
                        src(c + 1), buf(c + 1), gsems[1 - p]
                    )
                scatters[c] = pltpu.async_copy(buf(c), dst(c), ssems[p])
            scatters[n_chunks - 1].wait()

        return sc_copy_kernel(x2)

    return run(x2).reshape(shape)


def kernel(new_k, new_v):
    out_v = _sc_copy(new_v)
    out_k = _tc_copy(new_k)
    return (out_k, out_v)


# speedup vs baseline: 1.1525x; 1.1059x over previous
import jax
import jax.numpy as jnp
from jax.experimental import pallas as pl
from jax.experimental.pallas import tpu as pltpu

_HBLK = 8


def _copy_body(x_ref, o_ref):
    o_ref[...] = x_ref[...]


def _tc_copy(x):
    B, H, L, D = x.shape
    spec = pl.BlockSpec((1, _HBLK, L, D), lambda b, h: (b, h, 0, 0))
    return pl.pallas_call(
        _copy_body,
        grid=(B, H // _HBLK),
        in_specs=[spec],
        out_specs=spec,
        out_shape=jax.ShapeDtypeStruct(x.shape, x.dtype),
        compiler_params=pltpu.CompilerParams(
            dimension_semantics=("arbitrary", "arbitrary"),
        ),
    )(x)


def kernel(new_k, new_v):
    return (_tc_copy(new_k), _tc_copy(new_v))
